# baseline (speedup 1.0000x reference)
import functools

import jax
import jax.numpy as jnp
from jax import lax
from jax.experimental import pallas as pl
from jax.experimental.pallas import tpu as pltpu
from jax.experimental.pallas import tpu_sc as plsc

NC = 2
NS = 16
NW = NC * NS
CH = 128
NBUF = 6


def _deg_body(srcd_hbm, degp_hbm, idx_v, deg_v, *, n_vec, nslot):
    c = lax.axis_index("c")
    s = lax.axis_index("s")
    w = s * NC + c
    pltpu.sync_copy(srcd_hbm.at[w], idx_v)
    zeros = jnp.zeros((16,), dtype=jnp.float32)
    ones = jnp.full((16,), 1.0, dtype=jnp.float32)

    def zbody(j, carry):
        deg_v[pl.ds(j * 16, 16)] = zeros
        return carry

    lax.fori_loop(0, nslot // 16, zbody, 0)

    def body(j, carry):
        idx = idx_v[j]
        plsc.addupdate_scatter(deg_v, [idx], ones)
        return carry

    lax.fori_loop(0, n_vec, body, 0)
    pltpu.sync_copy(deg_v, degp_hbm.at[w])


def _hop_body(x2_hbm, srcg_hbm, dsts_hbm, zero_hbm, out_hbm,
              idx_s, idx_d, rows, acc, gsem, ssem,
              *, nchunk, stripe):
    c = lax.axis_index("c")
    s = lax.axis_index("s")
    pltpu.sync_copy(zero_hbm.at[pl.ds(s * stripe, stripe)],
                    acc.at[pl.ds(s * stripe, stripe)])
    pltpu.sync_copy(srcg_hbm.at[s], idx_s)
    pltpu.sync_copy(dsts_hbm.at[s], idx_d)
    plsc.subcore_barrier()

    xc = x2_hbm.at[c]

    def gather_desc(chunk, b):
        return pltpu.make_async_copy(xc.at[idx_s.at[chunk]], rows.at[b],
                                     gsem.at[b])

    def scatter_start(chunk, b):
        pltpu.async_copy(rows.at[b], acc.at[idx_d.at[chunk]], ssem.at[b],
                         add=True)

    def scatter_wait(chunk, b):
        pltpu.make_async_copy(rows.at[b], acc.at[idx_d.at[chunk]],
                              ssem.at[b]).wait()

    for b in range(NBUF - 1):
        gather_desc(b, b).start()

    def body(i, carry):
        base = i * NBUF
        for b in range(NBUF):
            j = base + b
            bm = (b - 1) % NBUF
            gather_desc(j, b).wait()
            scatter_start(j, b)
            @pl.when(j + NBUF - 1 < nchunk)
            def _refill():
                @pl.when(j > 0)
                def _wait_prev():
                    scatter_wait(j - 1, bm)
                gather_desc(j + NBUF - 1, bm).start()
        return carry

    lax.fori_loop(0, nchunk // NBUF, body, 0)
    for b in range(NBUF):
        scatter_wait(nchunk - NBUF + b, b)
    plsc.subcore_barrier()
    pltpu.sync_copy(acc.at[pl.ds(s * stripe, stripe)],
                    out_hbm.at[c, pl.ds(s * stripe, stripe)])


def _norm_body(degp_ref, out_ref):
    d = jnp.sum(degp_ref[...], axis=0)
    out_ref[...] = lax.rsqrt(jnp.maximum(d, 1.0))


def _scale_body(f_ref, n_ref, o_ref):
    o_ref[...] = (f_ref[...] * n_ref[...][:, :, None]).swapaxes(0, 1)


def _mid_body(a_ref, n_ref, h_ref, y_ref):
    nm = n_ref[...]
    a = a_ref[...]
    h_ref[...] = jnp.concatenate([a[0], a[1]], axis=1) * nm
    y_ref[...] = a * (nm * nm)[None, :, :]


def _fin_body(f_ref, h1_ref, a_ref, n_ref, w_ref, b_ref, o_ref):
    a = a_ref[...]
    h2 = jnp.concatenate([a[0], a[1]], axis=1) * n_ref[...]
    ssum = f_ref[...] + h1_ref[...] + h2
    y = jnp.dot(ssum, w_ref[...], preferred_element_type=jnp.float32)
    o_ref[...] = jnp.maximum(y + 3.0 * b_ref[...], 0.0)


def kernel(feat, edge_index, W, b):
    N, D = feat.shape
    HD = D // NC
    E = edge_index.shape[1]

    PWd = E // NW
    nvec_pad = -(-PWd // 16) * 16
    srcd = jnp.pad(edge_index[0].reshape(NW, PWd),
                   ((0, 0), (0, nvec_pad - PWd)),
                   constant_values=N).reshape(NW, nvec_pad // 16, 16)

    PWh = E // NS
    nchunk = -(-PWh // CH)
    nchunk = ((nchunk + NBUF - 1) // NBUF) * NBUF
    pad_h = nchunk * CH - PWh
    src = edge_index[0].reshape(NS, PWh)
    dst = edge_index[1].reshape(NS, PWh)
    srcg = jnp.pad(src, ((0, 0), (0, pad_h))).reshape(NS, nchunk, CH)
    dsts = jnp.pad(dst, ((0, 0), (0, pad_h)),
                   constant_values=N).reshape(NS, nchunk, CH)

    nslot = ((N + 1 + 127) // 128) * 128
    accr = -(-(N + 1) // (NS * 8)) * (NS * 8)
    stripe = accr // NS
    zero = jnp.zeros((accr, HD), dtype=jnp.float32)

    mesh = plsc.VectorSubcoreMesh(core_axis_name="c", subcore_axis_name="s")

    deg_call = pl.kernel(
        functools.partial(_deg_body, n_vec=nvec_pad // 16, nslot=nslot),
        out_type=jax.ShapeDtypeStruct((NW, nslot), jnp.float32),
        mesh=mesh,
        scratch_types=[
            pltpu.VMEM((nvec_pad // 16, 16), jnp.int32),
            pltpu.VMEM((nslot,), jnp.float32),
        ],
        compiler_params=pltpu.CompilerParams(needs_layout_passes=False),
    )
    degp = deg_call(srcd)

    hop_call = pl.kernel(
        functools.partial(_hop_body, nchunk=nchunk, stripe=stripe),
        out_type=jax.ShapeDtypeStruct((NC, accr, HD), jnp.float32),
        mesh=mesh,
        scratch_types=[
            pltpu.VMEM((nchunk, CH), jnp.int32),
            pltpu.VMEM((nchunk, CH), jnp.int32),
            pltpu.VMEM((NBUF, CH, HD), jnp.float32),
            pltpu.VMEM_SHARED((accr, HD), jnp.float32),
            pltpu.SemaphoreType.DMA((NBUF,)),
            pltpu.SemaphoreType.DMA((NBUF,)),
        ],
        compiler_params=pltpu.CompilerParams(use_tc_tiling_on_sc=False),
    )

    norm2d = pl.pallas_call(
        _norm_body,
        out_shape=jax.ShapeDtypeStruct((nslot // 128, 128), jnp.float32),
    )(degp.reshape(NW, nslot // 128, 128))
    normcol = norm2d.reshape(nslot)[:N][:, None]

    R = 5
    B = N // R
    row_spec = pl.BlockSpec((B, D), lambda i: (i, 0))
    col_spec = pl.BlockSpec((B, 1), lambda i: (i, 0))
    half_in_spec = pl.BlockSpec((B, NC, HD), lambda i: (i, 0, 0))
    half_out_spec = pl.BlockSpec((NC, B, HD), lambda i: (0, i, 0))
    w_spec = pl.BlockSpec((D, D), lambda i: (0, 0))
    b_spec = pl.BlockSpec((1, D), lambda i: (0, 0))

    x1h = pl.pallas_call(
        _scale_body,
        grid=(R,),
        in_specs=[half_in_spec, col_spec],
        out_specs=half_out_spec,
        out_shape=jax.ShapeDtypeStruct((NC, N, HD), jnp.float32),
    )(feat.reshape(N, NC, HD), normcol)

    aggp1 = hop_call(x1h, srcg, dsts, zero)

    h1, y1h = pl.pallas_call(
        _mid_body,
        grid=(R,),
        in_specs=[half_out_spec, col_spec],
        out_specs=[row_spec, half_out_spec],
        out_shape=[jax.ShapeDtypeStruct((N, D), jnp.float32),
                   jax.ShapeDtypeStruct((NC, N, HD), jnp.float32)],
    )(aggp1, normcol)

    aggp2 = hop_call(y1h, srcg, dsts, zero)

    rst = pl.pallas_call(
        _fin_body,
        grid=(R,),
        in_specs=[row_spec, row_spec, half_out_spec, col_spec, w_spec,
                  b_spec],
        out_specs=row_spec,
        out_shape=jax.ShapeDtypeStruct((N, D), jnp.float32),
    )(feat, h1, aggp2, normcol, W, b.reshape(1, D))

    return rst


# speedup vs baseline: 1.0840x; 1.0840x over previous
import functools

import jax
import jax.numpy as jnp
from jax import lax
from jax.experimental import pallas as pl
from jax.experimental.pallas import tpu as pltpu
from jax.experimental.pallas import tpu_sc as plsc

NC = 2
NS = 16
NW = NC * NS
CH = 320
NBUF = 2


def _deg_body(srcd_hbm, degp_hbm, idx_v, deg_v, *, n_vec, nslot):
    c = lax.axis_index("c")
    s = lax.axis_index("s")
    w = s * NC + c
    pltpu.sync_copy(srcd_hbm.at[w], idx_v)
    zeros = jnp.zeros((16,), dtype=jnp.float32)
    ones = jnp.full((16,), 1.0, dtype=jnp.float32)

    def zbody(j, carry):
        deg_v[pl.ds(j * 16, 16)] = zeros
        return carry

    lax.fori_loop(0, nslot // 16, zbody, 0)

    def body(j, carry):
        idx = idx_v[j]
        plsc.addupdate_scatter(deg_v, [idx], ones)
        return carry

    lax.fori_loop(0, n_vec, body, 0)
    pltpu.sync_copy(deg_v, degp_hbm.at[w])


def _hop_body(x2_hbm, srcg_hbm, dsts_hbm, zero_hbm, out_hbm,
              idx_s, idx_d, rows, acc, gsem,
              *, nchunk, stripe):
    c = lax.axis_index("c")
    s = lax.axis_index("s")
    pltpu.sync_copy(zero_hbm.at[pl.ds(s * stripe, stripe)],
                    acc.at[pl.ds(s * stripe, stripe)])
    pltpu.sync_copy(srcg_hbm.at[s], idx_s)
    pltpu.sync_copy(dsts_hbm.at[s], idx_d)
    plsc.subcore_barrier()

    xc = x2_hbm.at[c]

    def body(i, carry):
        base = i * NBUF
        cps = [pltpu.async_copy(xc.at[idx_s.at[base + b]], rows.at[b],
                                gsem.at[b])
               for b in range(NBUF)]
        for b in range(NBUF):
            cps[b].wait()
            pltpu.sync_copy(rows.at[b], acc.at[idx_d.at[base + b]], add=True)
        return carry

    lax.fori_loop(0, nchunk // NBUF, body, 0)
    plsc.subcore_barrier()
    pltpu.sync_copy(acc.at[pl.ds(s * stripe, stripe)],
                    out_hbm.at[c, pl.ds(s * stripe, stripe)])


def _norm_body(degp_ref, out_ref):
    d = jnp.sum(degp_ref[...], axis=0)
    out_ref[...] = lax.rsqrt(jnp.maximum(d, 1.0))


def _scale_body(f_ref, n_ref, o_ref):
    o_ref[...] = (f_ref[...] * n_ref[...][:, :, None]).swapaxes(0, 1)


def _mid_body(a_ref, n_ref, h_ref, y_ref):
    nm = n_ref[...]
    a = a_ref[...]
    h_ref[...] = jnp.concatenate([a[0], a[1]], axis=1) * nm
    y_ref[...] = a * (nm * nm)[None, :, :]


def _fin_body(f_ref, h1_ref, a_ref, n_ref, w_ref, b_ref, o_ref):
    a = a_ref[...]
    h2 = jnp.concatenate([a[0], a[1]], axis=1) * n_ref[...]
    ssum = f_ref[...] + h1_ref[...] + h2
    y = jnp.dot(ssum, w_ref[...], preferred_element_type=jnp.float32)
    o_ref[...] = jnp.maximum(y + 3.0 * b_ref[...], 0.0)


def kernel(feat, edge_index, W, b):
    N, D = feat.shape
    HD = D // NC
    E = edge_index.shape[1]

    PWd = E // NW
    nvec_pad = -(-PWd // 16) * 16
    srcd = jnp.pad(edge_index[0].reshape(NW, PWd),
                   ((0, 0), (0, nvec_pad - PWd)),
                   constant_values=N).reshape(NW, nvec_pad // 16, 16)

    PWh = E // NS
    OP = CH
    nchunk = -(-PWh // OP)
    nchunk = ((nchunk + NBUF - 1) // NBUF) * NBUF
    pad_h = nchunk * OP - PWh
    src = edge_index[0].reshape(NS, PWh)
    dst = edge_index[1].reshape(NS, PWh)
    srcg = jnp.pad(src, ((0, 0), (0, pad_h))).reshape(NS, nchunk, CH)
    dsts = jnp.pad(dst, ((0, 0), (0, pad_h)),
                   constant_values=N).reshape(NS, nchunk, CH)

    nslot = ((N + 1 + 127) // 128) * 128
    accr = -(-(N + 1) // (NS * 8)) * (NS * 8)
    stripe = accr // NS
    zero = jnp.zeros((accr, HD), dtype=jnp.float32)

    mesh = plsc.VectorSubcoreMesh(core_axis_name="c", subcore_axis_name="s")

    deg_call = pl.kernel(
        functools.partial(_deg_body, n_vec=nvec_pad // 16, nslot=nslot),
        out_type=jax.ShapeDtypeStruct((NW, nslot), jnp.float32),
        mesh=mesh,
        scratch_types=[
            pltpu.VMEM((nvec_pad // 16, 16), jnp.int32),
            pltpu.VMEM((nslot,), jnp.float32),
        ],
        compiler_params=pltpu.CompilerParams(needs_layout_passes=False),
    )
    degp = deg_call(srcd)

    hop_call = pl.kernel(
        functools.partial(_hop_body, nchunk=nchunk, stripe=stripe),
        out_type=jax.ShapeDtypeStruct((NC, accr, HD), jnp.float32),
        mesh=mesh,
        scratch_types=[
            pltpu.VMEM((nchunk, CH), jnp.int32),
            pltpu.VMEM((nchunk, CH), jnp.int32),
            pltpu.VMEM((NBUF, CH, HD), jnp.float32),
            pltpu.VMEM_SHARED((accr, HD), jnp.float32),
            pltpu.SemaphoreType.DMA((NBUF,)),
        ],
        compiler_params=pltpu.CompilerParams(use_tc_tiling_on_sc=False),
    )

    norm2d = pl.pallas_call(
        _norm_body,
        out_shape=jax.ShapeDtypeStruct((nslot // 128, 128), jnp.float32),
    )(degp.reshape(NW, nslot // 128, 128))
    normcol = norm2d.reshape(nslot)[:N][:, None]

    R = 5
    B = N // R
    row_spec = pl.BlockSpec((B, D), lambda i: (i, 0))
    col_spec = pl.BlockSpec((B, 1), lambda i: (i, 0))
    half_in_spec = pl.BlockSpec((B, NC, HD), lambda i: (i, 0, 0))
    half_out_spec = pl.BlockSpec((NC, B, HD), lambda i: (0, i, 0))
    w_spec = pl.BlockSpec((D, D), lambda i: (0, 0))
    b_spec = pl.BlockSpec((1, D), lambda i: (0, 0))

    x1h = pl.pallas_call(
        _scale_body,
        grid=(R,),
        in_specs=[half_in_spec, col_spec],
        out_specs=half_out_spec,
        out_shape=jax.ShapeDtypeStruct((NC, N, HD), jnp.float32),
    )(feat.reshape(N, NC, HD), normcol)

    aggp1 = hop_call(x1h, srcg, dsts, zero)

    h1, y1h = pl.pallas_call(
        _mid_body,
        grid=(R,),
        in_specs=[half_out_spec, col_spec],
        out_specs=[row_spec, half_out_spec],
        out_shape=[jax.ShapeDtypeStruct((N, D), jnp.float32),
                   jax.ShapeDtypeStruct((NC, N, HD), jnp.float32)],
    )(aggp1, normcol)

    aggp2 = hop_call(y1h, srcg, dsts, zero)

    rst = pl.pallas_call(
        _fin_body,
        grid=(R,),
        in_specs=[row_spec, row_spec, half_out_spec, col_spec, w_spec,
                  b_spec],
        out_specs=row_spec,
        out_shape=jax.ShapeDtypeStruct((N, D), jnp.float32),
    )(feat, h1, aggp2, normcol, W, b.reshape(1, D))

    return rst


# speedup vs baseline: 1.0980x; 1.0129x over previous
import functools

import jax
import jax.numpy as jnp
from jax import lax
from jax.experimental import pallas as pl
from jax.experimental.pallas import tpu as pltpu
from jax.experimental.pallas import tpu_sc as plsc

NC = 2
NS = 16
NW = NC * NS
CH = 320
NBUF = 2


def _deg_body(srcd_hbm, degp_hbm, idx_v, deg_v, *, n_vec, nslot):
    c = lax.axis_index("c")
    s = lax.axis_index("s")
    w = s * NC + c
    pltpu.sync_copy(srcd_hbm.at[w], idx_v)
    zeros = jnp.zeros((16,), dtype=jnp.float32)
    ones = jnp.full((16,), 1.0, dtype=jnp.float32)

    def zbody(j, carry):
        deg_v[pl.ds(j * 16, 16)] = zeros
        return carry

    lax.fori_loop(0, nslot // 16, zbody, 0)

    def body(j, carry):
        idx = idx_v[j]
        plsc.addupdate_scatter(deg_v, [idx], ones)
        return carry

    lax.fori_loop(0, n_vec, body, 0)
    pltpu.sync_copy(deg_v, degp_hbm.at[w])


def _hop_body(x2_hbm, srcg_hbm, dsts_hbm, zero_hbm, out_hbm,
              idx_s, idx_d, rows, acc, gsem, ssem,
              *, nchunk, stripe):
    c = lax.axis_index("c")
    s = lax.axis_index("s")
    pltpu.sync_copy(zero_hbm.at[pl.ds(s * stripe, stripe)],
                    acc.at[pl.ds(s * stripe, stripe)])
    pltpu.sync_copy(srcg_hbm.at[s], idx_s)
    pltpu.sync_copy(dsts_hbm.at[s], idx_d)
    plsc.subcore_barrier()

    xc = x2_hbm.at[c]

    def gwait(chunk, b):
        pltpu.make_async_copy(xc.at[idx_s.at[chunk]], rows.at[b],
                              gsem.at[b]).wait()

    def swait(chunk, b):
        pltpu.make_async_copy(rows.at[b], acc.at[idx_d.at[chunk]],
                              ssem.at[b]).wait()

    for b in range(NBUF):
        pltpu.async_copy(xc.at[idx_s.at[b]], rows.at[b], gsem.at[b])

    def body(i, carry):
        base = i * NBUF
        for b in range(NBUF):
            gwait(base + b, b)
            pltpu.async_copy(rows.at[b], acc.at[idx_d.at[base + b]],
                             ssem.at[b], add=True)
        for b in range(NBUF):
            j = base + b + NBUF
            @pl.when(j < nchunk)
            def _refill():
                swait(base + b, b)
                pltpu.async_copy(xc.at[idx_s.at[j]], rows.at[b], gsem.at[b])
        return carry

    lax.fori_loop(0, nchunk // NBUF, body, 0)
    for b in range(NBUF):
        swait(nchunk - NBUF + b, b)
    plsc.subcore_barrier()
    pltpu.sync_copy(acc.at[pl.ds(s * stripe, stripe)],
                    out_hbm.at[c, pl.ds(s * stripe, stripe)])


def _norm_body(degp_ref, out_ref):
    d = jnp.sum(degp_ref[...], axis=0)
    out_ref[...] = lax.rsqrt(jnp.maximum(d, 1.0))


def _scale_body(f_ref, n_ref, o_ref):
    o_ref[...] = (f_ref[...] * n_ref[...][:, :, None]).swapaxes(0, 1)


def _mid_body(a_ref, n_ref, h_ref, y_ref):
    nm = n_ref[...]
    a = a_ref[...]
    h_ref[...] = jnp.concatenate([a[0], a[1]], axis=1) * nm
    y_ref[...] = a * (nm * nm)[None, :, :]


def _fin_body(f_ref, h1_ref, a_ref, n_ref, w_ref, b_ref, o_ref):
    a = a_ref[...]
    h2 = jnp.concatenate([a[0], a[1]], axis=1) * n_ref[...]
    ssum = f_ref[...] + h1_ref[...] + h2
    y = jnp.dot(ssum, w_ref[...], preferred_element_type=jnp.float32)
    o_ref[...] = jnp.maximum(y + 3.0 * b_ref[...], 0.0)


def kernel(feat, edge_index, W, b):
    N, D = feat.shape
    HD = D // NC
    E = edge_index.shape[1]

    PWd = E // NW
    nvec_pad = -(-PWd // 16) * 16
    srcd = jnp.pad(edge_index[0].reshape(NW, PWd),
                   ((0, 0), (0, nvec_pad - PWd)),
                   constant_values=N).reshape(NW, nvec_pad // 16, 16)

    PWh = E // NS
    OP = CH
    nchunk = -(-PWh // OP)
    nchunk = ((nchunk + NBUF - 1) // NBUF) * NBUF
    pad_h = nchunk * OP - PWh
    src = edge_index[0].reshape(NS, PWh)
    dst = edge_index[1].reshape(NS, PWh)
    srcg = jnp.pad(src, ((0, 0), (0, pad_h))).reshape(NS, nchunk, CH)
    dsts = jnp.pad(dst, ((0, 0), (0, pad_h)),
                   constant_values=N).reshape(NS, nchunk, CH)

    nslot = ((N + 1 + 127) // 128) * 128
    accr = -(-(N + 1) // (NS * 8)) * (NS * 8)
    stripe = accr // NS
    zero = jnp.zeros((accr, HD), dtype=jnp.float32)

    mesh = plsc.VectorSubcoreMesh(core_axis_name="c", subcore_axis_name="s")

    deg_call = pl.kernel(
        functools.partial(_deg_body, n_vec=nvec_pad // 16, nslot=nslot),
        out_type=jax.ShapeDtypeStruct((NW, nslot), jnp.float32),
        mesh=mesh,
        scratch_types=[
            pltpu.VMEM((nvec_pad // 16, 16), jnp.int32),
            pltpu.VMEM((nslot,), jnp.float32),
        ],
        compiler_params=pltpu.CompilerParams(needs_layout_passes=False),
    )
    degp = deg_call(srcd)

    hop_call = pl.kernel(
        functools.partial(_hop_body, nchunk=nchunk, stripe=stripe),
        out_type=jax.ShapeDtypeStruct((NC, accr, HD), jnp.float32),
        mesh=mesh,
        scratch_types=[
            pltpu.VMEM((nchunk, CH), jnp.int32),
            pltpu.VMEM((nchunk, CH), jnp.int32),
            pltpu.VMEM((NBUF, CH, HD), jnp.float32),
            pltpu.VMEM_SHARED((accr, HD), jnp.float32),
            pltpu.SemaphoreType.DMA((NBUF,)),
            pltpu.SemaphoreType.DMA((NBUF,)),
        ],
        compiler_params=pltpu.CompilerParams(use_tc_tiling_on_sc=False),
    )

    norm2d = pl.pallas_call(
        _norm_body,
        out_shape=jax.ShapeDtypeStruct((nslot // 128, 128), jnp.float32),
    )(degp.reshape(NW, nslot // 128, 128))
    normcol = norm2d.reshape(nslot)[:N][:, None]

    R = 5
    B = N // R
    row_spec = pl.BlockSpec((B, D), lambda i: (i, 0))
    col_spec = pl.BlockSpec((B, 1), lambda i: (i, 0))
    half_in_spec = pl.BlockSpec((B, NC, HD), lambda i: (i, 0, 0))
    half_out_spec = pl.BlockSpec((NC, B, HD), lambda i: (0, i, 0))
    w_spec = pl.BlockSpec((D, D), lambda i: (0, 0))
    b_spec = pl.BlockSpec((1, D), lambda i: (0, 0))

    x1h = pl.pallas_call(
        _scale_body,
        grid=(R,),
        in_specs=[half_in_spec, col_spec],
        out_specs=half_out_spec,
        out_shape=jax.ShapeDtypeStruct((NC, N, HD), jnp.float32),
    )(feat.reshape(N, NC, HD), normcol)

    aggp1 = hop_call(x1h, srcg, dsts, zero)

    h1, y1h = pl.pallas_call(
        _mid_body,
        grid=(R,),
        in_specs=[half_out_spec, col_spec],
        out_specs=[row_spec, half_out_spec],
        out_shape=[jax.ShapeDtypeStruct((N, D), jnp.float32),
                   jax.ShapeDtypeStruct((NC, N, HD), jnp.float32)],
    )(aggp1, normcol)

    aggp2 = hop_call(y1h, srcg, dsts, zero)

    rst = pl.pallas_call(
        _fin_body,
        grid=(R,),
        in_specs=[row_spec, row_spec, half_out_spec, col_spec, w_spec,
                  b_spec],
        out_specs=row_spec,
        out_shape=jax.ShapeDtypeStruct((N, D), jnp.float32),
    )(feat, h1, aggp2, normcol, W, b.reshape(1, D))

    return rst


# speedup vs baseline: 1.1120x; 1.0127x over previous
import functools

import jax
import jax.numpy as jnp
from jax import lax
from jax.experimental import pallas as pl
from jax.experimental.pallas import tpu as pltpu
from jax.experimental.pallas import tpu_sc as plsc

NC = 2
NS = 16
NW = NC * NS
CH = 320
NBUF = 2


def _deg_body(srcd_hbm, degp_hbm, idx_v, deg_v, *, n_vec, nslot):
    c = lax.axis_index("c")
    s = lax.axis_index("s")
    w = s * NC + c
    pltpu.sync_copy(srcd_hbm.at[w], idx_v)
    zeros = jnp.zeros((16,), dtype=jnp.float32)
    ones = jnp.full((16,), 1.0, dtype=jnp.float32)

    def zbody(j, carry):
        deg_v[pl.ds(j * 16, 16)] = zeros
        return carry

    lax.fori_loop(0, nslot // 16, zbody, 0)

    def body(j, carry):
        idx = idx_v[j]
        plsc.addupdate_scatter(deg_v, [idx], ones)
        return carry

    lax.fori_loop(0, n_vec, body, 0)
    pltpu.sync_copy(deg_v, degp_hbm.at[w])


def _edge_loop(src_hbm, idx_s, idx_d, rows, acc, gsem, ssem, nchunk):
    def gwait(chunk, b):
        pltpu.make_async_copy(src_hbm.at[idx_s.at[chunk]], rows.at[b],
                              gsem.at[b]).wait()

    def swait(chunk, b):
        pltpu.make_async_copy(rows.at[b], acc.at[idx_d.at[chunk]],
                              ssem.at[b]).wait()

    for b in range(NBUF):
        pltpu.async_copy(src_hbm.at[idx_s.at[b]], rows.at[b], gsem.at[b])

    def body(i, carry):
        base = i * NBUF
        for b in range(NBUF):
            gwait(base + b, b)
            pltpu.async_copy(rows.at[b], acc.at[idx_d.at[base + b]],
                             ssem.at[b], add=True)
        for b in range(NBUF):
            j = base + b + NBUF
            @pl.when(j < nchunk)
            def _refill():
                swait(base + b, b)
                pltpu.async_copy(src_hbm.at[idx_s.at[j]], rows.at[b],
                                 gsem.at[b])
        return carry

    lax.fori_loop(0, nchunk // NBUF, body, 0)
    for b in range(NBUF):
        swait(nchunk - NBUF + b, b)


def _conv_body(x2_hbm, srcg_hbm, dsts_hbm, zero_hbm, norm_hbm,
               agg1_hbm, agg2_hbm, y1_hbm,
               idx_s, idx_d, rows, normt, acc, gsem, ssem,
               *, nchunk, stripe):
    c = lax.axis_index("c")
    s = lax.axis_index("s")
    pltpu.sync_copy(zero_hbm.at[pl.ds(s * stripe, stripe)],
                    acc.at[pl.ds(s * stripe, stripe)])
    pltpu.sync_copy(srcg_hbm.at[s], idx_s)
    pltpu.sync_copy(dsts_hbm.at[s], idx_d)
    pltpu.sync_copy(norm_hbm.at[pl.ds(s * (stripe // 16), stripe // 16)],
                    normt)
    plsc.subcore_barrier()

    _edge_loop(x2_hbm.at[c], idx_s, idx_d, rows, acc, gsem, ssem, nchunk)
    plsc.subcore_barrier()

    half = stripe // 2
    for hb in range(2):
        row0 = s * stripe + hb * half
        buf = rows.at[hb]
        pltpu.sync_copy(acc.at[pl.ds(row0, half)], buf)
        pltpu.sync_copy(buf, agg1_hbm.at[c, pl.ds(row0, half)])

        def gbody(gr, carry):
            nvec = normt[hb * (half // 16) + gr]
            n2 = nvec * nvec
            for lane in range(16):
                nv = n2[lane]
                r = gr * 16 + lane
                for q in range(4):
                    sl = pl.ds(q * 16, 16)
                    buf[r, sl] = buf[r, sl] * nv
            return carry

        lax.fori_loop(0, half // 16, gbody, 0)
        pltpu.sync_copy(buf, y1_hbm.at[c, pl.ds(row0, half)])
        pltpu.sync_copy(zero_hbm.at[pl.ds(row0, half)],
                        acc.at[pl.ds(row0, half)])
    plsc.subcore_barrier()

    _edge_loop(y1_hbm.at[c], idx_s, idx_d, rows, acc, gsem, ssem, nchunk)
    plsc.subcore_barrier()
    pltpu.sync_copy(acc.at[pl.ds(s * stripe, stripe)],
                    agg2_hbm.at[c, pl.ds(s * stripe, stripe)])


def _norm_body(degp_ref, out_ref):
    d = jnp.sum(degp_ref[...], axis=0)
    out_ref[...] = lax.rsqrt(jnp.maximum(d, 1.0))


def _scale_body(f_ref, n_ref, o_ref):
    o_ref[...] = (f_ref[...] * n_ref[...][:, :, None]).swapaxes(0, 1)


def _fin_body(f_ref, a1_ref, a2_ref, n_ref, w_ref, b_ref, o_ref):
    nm = n_ref[...]
    a1 = a1_ref[...]
    a2 = a2_ref[...]
    h1 = jnp.concatenate([a1[0], a1[1]], axis=1) * nm
    h2 = jnp.concatenate([a2[0], a2[1]], axis=1) * nm
    ssum = f_ref[...] + h1 + h2
    y = jnp.dot(ssum, w_ref[...], preferred_element_type=jnp.float32)
    o_ref[...] = jnp.maximum(y + 3.0 * b_ref[...], 0.0)


def kernel(feat, edge_index, W, b):
    N, D = feat.shape
    HD = D // NC
    E = edge_index.shape[1]

    PWd = E // NW
    nvec_pad = -(-PWd // 16) * 16
    srcd = jnp.pad(edge_index[0].reshape(NW, PWd),
                   ((0, 0), (0, nvec_pad - PWd)),
                   constant_values=N).reshape(NW, nvec_pad // 16, 16)

    PWh = E // NS
    OP = CH
    nchunk = -(-PWh // OP)
    nchunk = ((nchunk + NBUF - 1) // NBUF) * NBUF
    pad_h = nchunk * OP - PWh
    src = edge_index[0].reshape(NS, PWh)
    dst = edge_index[1].reshape(NS, PWh)
    srcg = jnp.pad(src, ((0, 0), (0, pad_h))).reshape(NS, nchunk, CH)
    dsts = jnp.pad(dst, ((0, 0), (0, pad_h)),
                   constant_values=N).reshape(NS, nchunk, CH)

    accr = -(-(N + 1) // (NS * 16)) * (NS * 16)
    nslot = accr
    stripe = accr // NS
    zero = jnp.zeros((accr, HD), dtype=jnp.float32)

    mesh = plsc.VectorSubcoreMesh(core_axis_name="c", subcore_axis_name="s")

    deg_call = pl.kernel(
        functools.partial(_deg_body, n_vec=nvec_pad // 16, nslot=nslot),
        out_type=jax.ShapeDtypeStruct((NW, nslot), jnp.float32),
        mesh=mesh,
        scratch_types=[
            pltpu.VMEM((nvec_pad // 16, 16), jnp.int32),
            pltpu.VMEM((nslot,), jnp.float32),
        ],
        compiler_params=pltpu.CompilerParams(needs_layout_passes=False),
    )
    degp = deg_call(srcd)

    agg_sds = jax.ShapeDtypeStruct((NC, accr, HD), jnp.float32)
    conv_call = pl.kernel(
        functools.partial(_conv_body, nchunk=nchunk, stripe=stripe),
        out_type=(agg_sds, agg_sds, agg_sds),
        mesh=mesh,
        scratch_types=[
            pltpu.VMEM((nchunk, CH), jnp.int32),
            pltpu.VMEM((nchunk, CH), jnp.int32),
            pltpu.VMEM((NBUF, CH, HD), jnp.float32),
            pltpu.VMEM((stripe // 16, 16), jnp.float32),
            pltpu.VMEM_SHARED((accr, HD), jnp.float32),
            pltpu.SemaphoreType.DMA((NBUF,)),
            pltpu.SemaphoreType.DMA((NBUF,)),
        ],
        compiler_params=pltpu.CompilerParams(use_tc_tiling_on_sc=False),
    )

    norm2d = pl.pallas_call(
        _norm_body,
        out_shape=jax.ShapeDtypeStruct((nslot // 16, 16), jnp.float32),
    )(degp.reshape(NW, nslot // 16, 16))
    normcol = norm2d.reshape(nslot)[:N][:, None]

    R = 5
    B = N // R
    row_spec = pl.BlockSpec((B, D), lambda i: (i, 0))
    col_spec = pl.BlockSpec((B, 1), lambda i: (i, 0))
    half_in_spec = pl.BlockSpec((B, NC, HD), lambda i: (i, 0, 0))
    half_out_spec = pl.BlockSpec((NC, B, HD), lambda i: (0, i, 0))
    w_spec = pl.BlockSpec((D, D), lambda i: (0, 0))
    b_spec = pl.BlockSpec((1, D), lambda i: (0, 0))

    x1h = pl.pallas_call(
        _scale_body,
        grid=(R,),
        in_specs=[half_in_spec, col_spec],
        out_specs=half_out_spec,
        out_shape=jax.ShapeDtypeStruct((NC, N, HD), jnp.float32),
    )(feat.reshape(N, NC, HD), normcol)

    aggp1, aggp2, _y1 = conv_call(x1h, srcg, dsts, zero, norm2d)

    rst = pl.pallas_call(
        _fin_body,
        grid=(R,),
        in_specs=[row_spec, half_out_spec, half_out_spec, col_spec, w_spec,
                  b_spec],
        out_specs=row_spec,
        out_shape=jax.ShapeDtypeStruct((N, D), jnp.float32),
    )(feat, aggp1, aggp2, normcol, W, b.reshape(1, D))

    return rst


# speedup vs baseline: 1.1231x; 1.0100x over previous
import functools

import jax
import jax.numpy as jnp
from jax import lax
from jax.experimental import pallas as pl
from jax.experimental.pallas import tpu as pltpu
from jax.experimental.pallas import tpu_sc as plsc

NC = 2
NS = 16
CH = 256
NBUF = 2
NCK = 4


def _edge_loop(src_hbm, idx_s, idx_d, rows, acc, gsem, ssem, nchunk):
    def gwait(chunk, b):
        pltpu.make_async_copy(src_hbm.at[idx_s.at[chunk]], rows.at[b],
                              gsem.at[b]).wait()

    def swait(chunk, b):
        pltpu.make_async_copy(rows.at[b], acc.at[idx_d.at[chunk]],
                              ssem.at[b]).wait()

    for b in range(NBUF):
        pltpu.async_copy(src_hbm.at[idx_s.at[b]], rows.at[b], gsem.at[b])

    def body(i, carry):
        base = i * NBUF
        for b in range(NBUF):
            gwait(base + b, b)
            pltpu.async_copy(rows.at[b], acc.at[idx_d.at[base + b]],
                             ssem.at[b], add=True)
        for b in range(NBUF):
            j = base + b + NBUF
            @pl.when(j < nchunk)
            def _refill():
                swait(base + b, b)
                pltpu.async_copy(src_hbm.at[idx_s.at[j]], rows.at[b],
                                 gsem.at[b])
        return carry

    lax.fori_loop(0, nchunk // NBUF, body, 0)
    for b in range(NBUF):
        swait(nchunk - NBUF + b, b)


def _scale_stripe(buf, normt, ck, chk, power):
    def gbody(gr, carry):
        nvec = normt[ck * (chk // 16) + gr]
        nv16 = nvec * nvec if power == 2 else nvec
        for lane in range(16):
            nv = nv16[lane]
            r = gr * 16 + lane
            for q in range(4):
                sl = pl.ds(q * 16, 16)
                buf[r, sl] = buf[r, sl] * nv
        return carry

    lax.fori_loop(0, chk // 16, gbody, 0)


def _conv_body(featp_hbm, srcg_hbm, dsts_hbm, zero_hbm, zerod_hbm,
               agg1_hbm, agg2_hbm, y1_hbm, x1_hbm, norm_hbm,
               idx_s, idx_d, rows, deg_v, ident_v, normt, acc, degacc,
               gsem, ssem, *, nchunk, stripe):
    c = lax.axis_index("c")
    s = lax.axis_index("s")
    nrow = stripe // 16
    chk = stripe // NCK

    pltpu.sync_copy(zero_hbm.at[pl.ds(s * stripe, stripe)],
                    acc.at[pl.ds(s * stripe, stripe)])
    pltpu.sync_copy(zerod_hbm.at[pl.ds(s * nrow, nrow)],
                    degacc.at[pl.ds(s * nrow, nrow)])
    pltpu.sync_copy(srcg_hbm.at[s], idx_s)
    pltpu.sync_copy(dsts_hbm.at[s], idx_d)

    zeros16 = jnp.zeros((16,), dtype=jnp.float32)
    ones16 = jnp.full((16,), 1.0, dtype=jnp.float32)
    iota16 = lax.iota(jnp.int32, 16)

    def zdeg(j, carry):
        deg_v[j] = zeros16
        return carry

    lax.fori_loop(0, nrow * NS, zdeg, 0)

    def hist2(i, carry):
        idx = idx_s[i // (CH // 16), pl.ds((i % (CH // 16)) * 16, 16)]
        plsc.addupdate_scatter(deg_v,
                               [lax.shift_right_logical(idx, 4),
                                lax.bitwise_and(idx, 15)],
                               ones16)
        return carry

    lax.fori_loop(0, nchunk * (CH // 16), hist2, 0)

    def idbody(j, carry):
        ident_v[pl.ds(j * 16, 16)] = iota16 + j * 16
        return carry

    lax.fori_loop(0, (nrow * NS) // 16, idbody, 0)
    plsc.subcore_barrier()
    pltpu.sync_copy(deg_v, degacc.at[ident_v], add=True)
    plsc.subcore_barrier()

    pltpu.sync_copy(degacc.at[pl.ds(s * nrow, nrow)], normt)

    def nbody(r, carry):
        x = jnp.maximum(normt[r], 1.0)
        i = plsc.bitcast(x, jnp.int32)
        i = 0x5F3759DF - lax.shift_right_logical(i, 1)
        y = plsc.bitcast(i, jnp.float32)
        y = y * (1.5 - 0.5 * x * y * y)
        y = y * (1.5 - 0.5 * x * y * y)
        y = y * (1.5 - 0.5 * x * y * y)
        normt[r] = y
        return carry

    lax.fori_loop(0, nrow, nbody, 0)

    @pl.when(c == 0)
    def _write_norm():
        pltpu.sync_copy(normt, norm_hbm.at[pl.ds(s * nrow, nrow)])

    for ck in range(NCK):
        row0 = s * stripe + ck * chk
        buf = rows.at[ck % NBUF].at[pl.ds(0, chk)]
        pltpu.sync_copy(featp_hbm.at[c, pl.ds(row0, chk)], buf)
        _scale_stripe(buf, normt, ck, chk, 1)
        pltpu.sync_copy(buf, x1_hbm.at[c, pl.ds(row0, chk)])
    plsc.subcore_barrier()

    _edge_loop(x1_hbm.at[c], idx_s, idx_d, rows, acc, gsem, ssem, nchunk)
    plsc.subcore_barrier()

    for ck in range(NCK):
        row0 = s * stripe + ck * chk
        buf = rows.at[ck % NBUF].at[pl.ds(0, chk)]
        pltpu.sync_copy(acc.at[pl.ds(row0, chk)], buf)
        pltpu.sync_copy(buf, agg1_hbm.at[c, pl.ds(row0, chk)])
        _scale_stripe(buf, normt, ck, chk, 2)
        pltpu.sync_copy(buf, y1_hbm.at[c, pl.ds(row0, chk)])
        pltpu.sync_copy(zero_hbm.at[pl.ds(row0, chk)],
                        acc.at[pl.ds(row0, chk)])
    plsc.subcore_barrier()

    _edge_loop(y1_hbm.at[c], idx_s, idx_d, rows, acc, gsem, ssem, nchunk)
    plsc.subcore_barrier()
    pltpu.sync_copy(acc.at[pl.ds(s * stripe, stripe)],
                    agg2_hbm.at[c, pl.ds(s * stripe, stripe)])


def _fin_body(f_ref, a1_ref, a2_ref, n_ref, w_ref, b_ref, o_ref):
    nm = n_ref[...]
    a1 = a1_ref[...]
    a2 = a2_ref[...]
    h1 = jnp.concatenate([a1[0], a1[1]], axis=1) * nm
    h2 = jnp.concatenate([a2[0], a2[1]], axis=1) * nm
    ssum = f_ref[...] + h1 + h2
    y = jnp.dot(ssum, w_ref[...], preferred_element_type=jnp.float32)
    o_ref[...] = jnp.maximum(y + 3.0 * b_ref[...], 0.0)


def kernel(feat, edge_index, W, b):
    N, D = feat.shape
    HD = D // NC
    E = edge_index.shape[1]

    accr = -(-(N + 1) // (NS * 16)) * (NS * 16)
    stripe = accr // NS

    PWh = E // NS
    nchunk = -(-PWh // CH)
    nchunk = ((nchunk + NBUF - 1) // NBUF) * NBUF
    pad_h = nchunk * CH - PWh
    srcg = jnp.pad(edge_index[0].reshape(NS, PWh), ((0, 0), (0, pad_h)),
                   constant_values=N).reshape(NS, nchunk, CH)
    dsts = jnp.pad(edge_index[1].reshape(NS, PWh), ((0, 0), (0, pad_h)),
                   constant_values=N).reshape(NS, nchunk, CH)

    featp = jnp.pad(jnp.swapaxes(feat.reshape(N, NC, HD), 0, 1),
                    ((0, 0), (0, accr - N), (0, 0)))
    zero = jnp.zeros((accr, HD), dtype=jnp.float32)
    zerod = jnp.zeros((accr // 16, 16), dtype=jnp.float32)

    mesh = plsc.VectorSubcoreMesh(core_axis_name="c", subcore_axis_name="s")
    agg_sds = jax.ShapeDtypeStruct((NC, accr, HD), jnp.float32)

    conv_call = pl.kernel(
        functools.partial(_conv_body, nchunk=nchunk, stripe=stripe),
        out_type=(agg_sds, agg_sds, agg_sds, agg_sds,
                  jax.ShapeDtypeStruct((accr // 16, 16), jnp.float32)),
        mesh=mesh,
        scratch_types=[
            pltpu.VMEM((nchunk, CH), jnp.int32),
            pltpu.VMEM((nchunk, CH), jnp.int32),
            pltpu.VMEM((NBUF, CH, HD), jnp.float32),
            pltpu.VMEM((accr // 16, 16), jnp.float32),
            pltpu.VMEM((accr // 16,), jnp.int32),
            pltpu.VMEM((stripe // 16, 16), jnp.float32),
            pltpu.VMEM_SHARED((accr, HD), jnp.float32),
            pltpu.VMEM_SHARED((accr // 16, 16), jnp.float32),
            pltpu.SemaphoreType.DMA((NBUF,)),
            pltpu.SemaphoreType.DMA((NBUF,)),
        ],
        compiler_params=pltpu.CompilerParams(use_tc_tiling_on_sc=False,
                                             needs_layout_passes=False),
    )
    aggp1, aggp2, _y1, _x1, norm2d = conv_call(featp, srcg, dsts, zero,
                                               zerod)
    normcol = norm2d.reshape(accr)[:N][:, None]

    R = 5
    B = N // R
    row_spec = pl.BlockSpec((B, D), lambda i: (i, 0))
    col_spec = pl.BlockSpec((B, 1), lambda i: (i, 0))
    half_spec = pl.BlockSpec((NC, B, HD), lambda i: (0, i, 0))
    w_spec = pl.BlockSpec((D, D), lambda i: (0, 0))
    b_spec = pl.BlockSpec((1, D), lambda i: (0, 0))

    rst = pl.pallas_call(
        _fin_body,
        grid=(R,),
        in_specs=[row_spec, half_spec, half_spec, col_spec, w_spec, b_spec],
        out_specs=row_spec,
        out_shape=jax.ShapeDtypeStruct((N, D), jnp.float32),
    )(feat, aggp1, aggp2, normcol, W, b.reshape(1, D))

    return rst
